# baseline (speedup 1.0000x reference)
import functools

import jax
import jax.numpy as jnp
from jax import lax
from jax.experimental import pallas as pl
from jax.experimental.pallas import tpu as pltpu
from jax.experimental.pallas import tpu_sc as plsc

N = 10000
E = 320000
DIN = 128
H = 8
D = 16
HD = H * D
C = 16

NP = 10240
EP = 327680
NC = 2
NS = 16
NW = NC * NS
EW = EP // NW
K = 128
NCHUNK = EW // K
U = 10
NSUPER = NCHUNK // U
RPT = NP // NS

f32 = jnp.float32
i32 = jnp.int32

_HIGH = jax.lax.Precision.HIGHEST


def _sc_mesh():
    return plsc.VectorSubcoreMesh(
        core_axis_name="c", subcore_axis_name="s",
        num_cores=NC, num_subcores=NS)


_BR = 1024


def _dense1_body(x_ref, w_ref, al_ref, ar_ref,
                 z_ref, el_ref, er_ref, mxl_ref, mxr_ref):
    i = pl.program_id(0)
    z = jnp.dot(x_ref[...], w_ref[...], preferred_element_type=f32,
                precision=_HIGH)
    z_ref[...] = z
    el = jnp.dot(z, al_ref[...], preferred_element_type=f32, precision=_HIGH)
    er = jnp.dot(z, ar_ref[...], preferred_element_type=f32, precision=_HIGH)
    el_ref[...] = el
    er_ref[...] = er

    @pl.when(i == 0)
    def _():
        mxl_ref[...] = jnp.full((1, 16), -1e30, f32)
        mxr_ref[...] = jnp.full((1, 16), -1e30, f32)

    mxl_ref[...] = jnp.maximum(mxl_ref[...], jnp.max(el, axis=0, keepdims=True))
    mxr_ref[...] = jnp.maximum(mxr_ref[...], jnp.max(er, axis=0, keepdims=True))


def _dense1(xp, W, AL, AR):
    return pl.pallas_call(
        _dense1_body,
        grid=(NP // _BR,),
        in_specs=[pl.BlockSpec((_BR, HD), lambda i: (i, 0)),
                  pl.BlockSpec((HD, HD), lambda i: (0, 0)),
                  pl.BlockSpec((HD, 16), lambda i: (0, 0)),
                  pl.BlockSpec((HD, 16), lambda i: (0, 0))],
        out_specs=[pl.BlockSpec((_BR, HD), lambda i: (i, 0)),
                   pl.BlockSpec((_BR, 16), lambda i: (i, 0)),
                   pl.BlockSpec((_BR, 16), lambda i: (i, 0)),
                   pl.BlockSpec((1, 16), lambda i: (0, 0)),
                   pl.BlockSpec((1, 16), lambda i: (0, 0))],
        out_shape=[jax.ShapeDtypeStruct((NP, HD), f32),
                   jax.ShapeDtypeStruct((NP, 16), f32),
                   jax.ShapeDtypeStruct((NP, 16), f32),
                   jax.ShapeDtypeStruct((1, 16), f32),
                   jax.ShapeDtypeStruct((1, 16), f32)],
    )(xp, W, AL, AR)


def _dense2_body(p0_ref, p1_ref, d0_ref, d1_ref, rx_ref, b_ref, w_ref,
                 al_ref, ar_ref,
                 z_ref, el_ref, er_ref, mxl_ref, mxr_ref):
    i = pl.program_id(0)
    rd = 1.0 / jnp.maximum(d0_ref[...] + d1_ref[...], 1e-9)
    rde = jnp.dot(rd, rx_ref[...], preferred_element_type=f32,
                  precision=_HIGH)
    h = jnp.maximum((p0_ref[...] + p1_ref[...]) * rde + b_ref[...], 0.0)
    z = jnp.dot(h, w_ref[...], preferred_element_type=f32, precision=_HIGH)
    z_ref[...] = z
    el = jnp.dot(z, al_ref[...], preferred_element_type=f32, precision=_HIGH)
    er = jnp.dot(z, ar_ref[...], preferred_element_type=f32, precision=_HIGH)
    el_ref[...] = el
    er_ref[...] = er

    @pl.when(i == 0)
    def _():
        mxl_ref[...] = jnp.full((1, 16), -1e30, f32)
        mxr_ref[...] = jnp.full((1, 16), -1e30, f32)

    mxl_ref[...] = jnp.maximum(mxl_ref[...], jnp.max(el, axis=0, keepdims=True))
    mxr_ref[...] = jnp.maximum(mxr_ref[...], jnp.max(er, axis=0, keepdims=True))


def _dense2(h0, h1p, d0, d1, rexp, brow, W, AL, AR):
    return pl.pallas_call(
        _dense2_body,
        grid=(NP // _BR,),
        in_specs=[pl.BlockSpec((_BR, HD), lambda i: (i, 0)),
                  pl.BlockSpec((_BR, HD), lambda i: (i, 0)),
                  pl.BlockSpec((_BR, 16), lambda i: (i, 0)),
                  pl.BlockSpec((_BR, 16), lambda i: (i, 0)),
                  pl.BlockSpec((16, HD), lambda i: (0, 0)),
                  pl.BlockSpec((1, HD), lambda i: (0, 0)),
                  pl.BlockSpec((HD, HD), lambda i: (0, 0)),
                  pl.BlockSpec((HD, 16), lambda i: (0, 0)),
                  pl.BlockSpec((HD, 16), lambda i: (0, 0))],
        out_specs=[pl.BlockSpec((_BR, HD), lambda i: (i, 0)),
                   pl.BlockSpec((_BR, 16), lambda i: (i, 0)),
                   pl.BlockSpec((_BR, 16), lambda i: (i, 0)),
                   pl.BlockSpec((1, 16), lambda i: (0, 0)),
                   pl.BlockSpec((1, 16), lambda i: (0, 0))],
        out_shape=[jax.ShapeDtypeStruct((NP, HD), f32),
                   jax.ShapeDtypeStruct((NP, 16), f32),
                   jax.ShapeDtypeStruct((NP, 16), f32),
                   jax.ShapeDtypeStruct((1, 16), f32),
                   jax.ShapeDtypeStruct((1, 16), f32)],
    )(h0, h1p, d0, d1, rexp, brow, W, AL, AR)


def _final_body(w0_ref, w1_ref, z2_ref, rexp_ref, b2_ref, wc_ref, bc_ref,
                o_ref):
    w = w0_ref[...] + w1_ref[...]
    rows = lax.broadcasted_iota(i32, (NP, 1), 0)
    w = jnp.where(rows < N, w, 0.0)
    we = jnp.dot(w, rexp_ref[...], preferred_element_type=f32,
                 precision=_HIGH)
    s = jnp.sum(we * z2_ref[...], axis=0, keepdims=True)
    hg = s * (1.0 / N) + b2_ref[...]
    o_ref[...] = jnp.dot(hg, wc_ref[...], preferred_element_type=f32,
                         precision=_HIGH) + bc_ref[...]


def _final(wp, z2, rexp, b2row, Wc, bcrow):
    return pl.pallas_call(
        _final_body,
        out_shape=jax.ShapeDtypeStruct((1, C), f32),
    )(wp[:NP], wp[NP:], z2, rexp, b2row, Wc, bcrow)


@functools.partial(
    pl.kernel,
    out_type=[jax.ShapeDtypeStruct((EP, 16), f32),
              jax.ShapeDtypeStruct((2 * NP, 16), f32)],
    mesh=_sc_mesh(),
    compiler_params=pltpu.CompilerParams(use_tc_tiling_on_sc=False),
    scratch_types=[
        pltpu.VMEM((K,), i32),
        pltpu.VMEM((K,), i32),
        pltpu.VMEM((K, 16), f32),
        pltpu.VMEM((K, 16), f32),
        pltpu.VMEM((K, 16), f32),
        pltpu.VMEM((16,), f32),
        pltpu.VMEM((RPT, 16), f32),
        pltpu.VMEM_SHARED((NP, 16), f32),
        pltpu.VMEM_SHARED((NP, 16), f32),
        pltpu.VMEM_SHARED((NP, 16), f32),
        pltpu.SemaphoreType.DMA,
        pltpu.SemaphoreType.DMA,
    ],
)
def _edge_stats(src_hbm, dst_hbm, el_hbm, er_hbm, b_hbm,
                ee_hbm, dp_hbm,
                srcv, dstv, elv, erv, eev, bv, zbuf, dacc, elsh, ersh,
                sem1, sem2):
    c = lax.axis_index("c")
    s = lax.axis_index("s")
    wid = s * NC + c
    base = wid * EW

    pltpu.sync_copy(b_hbm, bv)
    pltpu.sync_copy(el_hbm.at[pl.ds(s * RPT, RPT)],
                    elsh.at[pl.ds(s * RPT, RPT)])
    pltpu.sync_copy(er_hbm.at[pl.ds(s * RPT, RPT)],
                    ersh.at[pl.ds(s * RPT, RPT)])

    def _zrow(j, carry):
        zbuf[j, :] = jnp.zeros((16,), f32)
        return carry

    lax.fori_loop(0, RPT, _zrow, 0, unroll=8)
    pltpu.sync_copy(zbuf, dacc.at[pl.ds(s * RPT, RPT)])
    plsc.subcore_barrier()

    bvec = bv[...]

    def _chunk(t, carry):
        off = base + t * K
        i1 = pltpu.async_copy(src_hbm.at[pl.ds(off, K)], srcv, sem1)
        i2 = pltpu.async_copy(dst_hbm.at[pl.ds(off, K)], dstv, sem2)
        i1.wait()
        i2.wait()
        g1 = pltpu.async_copy(elsh.at[srcv], elv, sem1)
        g2 = pltpu.async_copy(ersh.at[dstv], erv, sem2)
        g1.wait()
        g2.wait()

        def _row(r, rc):
            x = elv[r, :] + erv[r, :]
            e = jnp.maximum(x, 0.2 * x)
            eev[r, :] = jnp.exp(e - bvec)
            return rc

        lax.fori_loop(0, K, _row, 0, unroll=4)
        w1 = pltpu.async_copy(eev, ee_hbm.at[pl.ds(off, K)], sem1)
        pltpu.sync_copy(eev, dacc.at[dstv], add=True)
        w1.wait()
        return carry

    lax.fori_loop(0, NCHUNK, _chunk, 0)
    plsc.subcore_barrier()
    pltpu.sync_copy(dacc.at[pl.ds(s * RPT, RPT)],
                    dp_hbm.at[pl.ds(c * NP + s * RPT, RPT)])


@functools.partial(
    pl.kernel,
    out_type=jax.ShapeDtypeStruct((2 * NP, HD), f32),
    mesh=_sc_mesh(),
    compiler_params=pltpu.CompilerParams(use_tc_tiling_on_sc=False),
    scratch_types=[
        pltpu.VMEM((K,), i32),
        pltpu.VMEM((K,), i32),
        pltpu.VMEM((K, 16), f32),
        pltpu.VMEM((K, HD), f32),
        pltpu.VMEM_SHARED((NP, HD), f32),
        pltpu.SemaphoreType.DMA,
        pltpu.SemaphoreType.DMA,
    ],
)
def _aggregate(src_hbm, dst_hbm, ee_hbm, z_hbm,
               hp_hbm,
               srcv, dstv, eev, zrows, hacc, sem1, sem2):
    c = lax.axis_index("c")
    s = lax.axis_index("s")
    wid = s * NC + c
    base = wid * EW

    def _zrow(j, carry):
        zrows[j // 8, pl.ds((j % 8) * D, D)] = jnp.zeros((16,), f32)
        return carry

    lax.fori_loop(0, K * 8, _zrow, 0, unroll=8)

    def _zcp(j, carry):
        pltpu.sync_copy(zrows, hacc.at[pl.ds(s * RPT + j * K, K)])
        return carry

    lax.fori_loop(0, RPT // K, _zcp, 0)
    plsc.subcore_barrier()

    def _chunk(t, carry):
        off = base + t * K
        i1 = pltpu.async_copy(src_hbm.at[pl.ds(off, K)], srcv, sem1)
        i2 = pltpu.async_copy(dst_hbm.at[pl.ds(off, K)], dstv, sem2)
        i1.wait()
        i2.wait()
        g1 = pltpu.async_copy(z_hbm.at[srcv], zrows, sem1)
        g2 = pltpu.async_copy(ee_hbm.at[pl.ds(off, K)], eev, sem2)
        g2.wait()
        g1.wait()

        def _row(r, rc):
            av = eev[r, :]
            for h in range(H):
                zrows[r, pl.ds(h * D, D)] = zrows[r, pl.ds(h * D, D)] * av[h]
            return rc

        lax.fori_loop(0, K, _row, 0, unroll=4)
        pltpu.sync_copy(zrows, hacc.at[dstv], add=True)
        return carry

    lax.fori_loop(0, NCHUNK, _chunk, 0)
    plsc.subcore_barrier()

    def _out(j, carry):
        r0 = s * RPT + j * K
        pltpu.sync_copy(hacc.at[pl.ds(r0, K)],
                        hp_hbm.at[pl.ds(c * NP + r0, K)])
        return carry

    lax.fori_loop(0, RPT // K, _out, 0)


@functools.partial(
    pl.kernel,
    out_type=jax.ShapeDtypeStruct((2 * NP, 16), f32),
    mesh=_sc_mesh(),
    compiler_params=pltpu.CompilerParams(use_tc_tiling_on_sc=False),
    scratch_types=[
        pltpu.VMEM((K,), i32),
        pltpu.VMEM((K,), i32),
        pltpu.VMEM((K, 16), f32),
        pltpu.VMEM((K, 16), f32),
        pltpu.VMEM((RPT, 16), f32),
        pltpu.VMEM_SHARED((NP, 16), f32),
        pltpu.VMEM_SHARED((NP, 16), f32),
        pltpu.SemaphoreType.DMA,
        pltpu.SemaphoreType.DMA,
    ],
)
def _edge_weights(src_hbm, dst_hbm, ee_hbm, dp_hbm,
                  wp_hbm,
                  srcv, dstv, eev, rdv, zbuf, wacc, rdsh, sem1, sem2):
    c = lax.axis_index("c")
    s = lax.axis_index("s")
    wid = s * NC + c
    base = wid * EW

    def _rc(j, carry):
        r0 = s * RPT + j * K
        pltpu.sync_copy(dp_hbm.at[pl.ds(r0, K)], eev)
        pltpu.sync_copy(dp_hbm.at[pl.ds(NP + r0, K)], rdv)

        def _row(r, rc2):
            eev[r, :] = 1.0 / jnp.maximum(eev[r, :] + rdv[r, :], 1e-9)
            return rc2

        lax.fori_loop(0, K, _row, 0, unroll=4)
        pltpu.sync_copy(eev, rdsh.at[pl.ds(r0, K)])
        return carry

    lax.fori_loop(0, RPT // K, _rc, 0)

    def _zrow(j, carry):
        zbuf[j, :] = jnp.zeros((16,), f32)
        return carry

    lax.fori_loop(0, RPT, _zrow, 0, unroll=8)
    pltpu.sync_copy(zbuf, wacc.at[pl.ds(s * RPT, RPT)])
    plsc.subcore_barrier()

    def _chunk(t, carry):
        off = base + t * K
        i1 = pltpu.async_copy(src_hbm.at[pl.ds(off, K)], srcv, sem1)
        i2 = pltpu.async_copy(dst_hbm.at[pl.ds(off, K)], dstv, sem2)
        i1.wait()
        i2.wait()
        g1 = pltpu.async_copy(rdsh.at[dstv], rdv, sem1)
        g2 = pltpu.async_copy(ee_hbm.at[pl.ds(off, K)], eev, sem2)
        g2.wait()
        g1.wait()

        def _row(r, rc):
            eev[r, :] = eev[r, :] * rdv[r, :]
            return rc

        lax.fori_loop(0, K, _row, 0, unroll=4)
        pltpu.sync_copy(eev, wacc.at[srcv], add=True)
        return carry

    lax.fori_loop(0, NCHUNK, _chunk, 0)
    plsc.subcore_barrier()
    pltpu.sync_copy(wacc.at[pl.ds(s * RPT, RPT)],
                    wp_hbm.at[pl.ds(c * NP + s * RPT, RPT)])


def _amat(a):
    af = a.reshape(HD)
    cols = jnp.arange(HD) // D
    m = jnp.zeros((HD, 16), f32)
    m = m.at[jnp.arange(HD), cols].set(af)
    m = m.at[jnp.arange(HD), cols + 8].set(af)
    return m


def _lrelu(x):
    return jnp.maximum(x, 0.2 * x)


def kernel(feat, edge_index, W1, al1, ar1, b1, W2, al2, ar2, b2, Wc, bc):
    xp = jnp.zeros((NP, DIN), f32).at[:N].set(feat)
    pad_ids = N + (jnp.arange(EP - E, dtype=i32) % 16)
    srcp = jnp.concatenate([edge_index[0], pad_ids])
    dstp = jnp.concatenate([edge_index[1], pad_ids])
    AL1, AR1 = _amat(al1), _amat(ar1)
    AL2, AR2 = _amat(al2), _amat(ar2)
    cc = jnp.arange(HD)
    rexp = ((cc[None, :] // D) == (jnp.arange(16)[:, None] % 8)).astype(f32) * 0.5

    z1, el1, er1, mxl1, mxr1 = _dense1(xp, W1, AL1, AR1)
    b16_1 = _lrelu(mxl1 + mxr1).reshape(16)
    ee1, dp1 = _edge_stats(srcp, dstp, el1, er1, b16_1)
    h1p = _aggregate(srcp, dstp, ee1, z1)

    z2, el2, er2, mxl2, mxr2 = _dense2(
        h1p[:NP], h1p[NP:], dp1[:NP], dp1[NP:], rexp,
        b1.reshape(1, HD), W2, AL2, AR2)
    b16_2 = _lrelu(mxl2 + mxr2).reshape(16)
    ee2, dp2 = _edge_stats(srcp, dstp, el2, er2, b16_2)
    wp = _edge_weights(srcp, dstp, ee2, dp2)

    return _final(wp, z2, rexp, b2.reshape(1, HD), Wc, bc.reshape(1, C))


# speedup vs baseline: 1.2550x; 1.2550x over previous
import functools

import jax
import jax.numpy as jnp
from jax import lax
from jax.experimental import pallas as pl
from jax.experimental.pallas import tpu as pltpu
from jax.experimental.pallas import tpu_sc as plsc

N = 10000
E = 320000
DIN = 128
H = 8
D = 16
HD = H * D
C = 16

NP = 10240
EP = 327680
NC = 2
NS = 16
NW = NC * NS
EW = EP // NW
K = 128
NCHUNK = EW // K
U = 10
NSUPER = NCHUNK // U
RPT = NP // NS

f32 = jnp.float32
i32 = jnp.int32

_HIGH = jax.lax.Precision.HIGHEST


def _sc_mesh():
    return plsc.VectorSubcoreMesh(
        core_axis_name="c", subcore_axis_name="s",
        num_cores=NC, num_subcores=NS)


_BR = 1024


def _dense1_body(x_ref, w_ref, al_ref, ar_ref,
                 z_ref, el_ref, er_ref, mxl_ref, mxr_ref):
    i = pl.program_id(0)
    z = jnp.dot(x_ref[...], w_ref[...], preferred_element_type=f32,
                precision=_HIGH)
    z_ref[...] = z
    el = jnp.dot(z, al_ref[...], preferred_element_type=f32, precision=_HIGH)
    er = jnp.dot(z, ar_ref[...], preferred_element_type=f32, precision=_HIGH)
    el_ref[...] = el
    er_ref[...] = er

    @pl.when(i == 0)
    def _():
        mxl_ref[...] = jnp.full((1, 16), -1e30, f32)
        mxr_ref[...] = jnp.full((1, 16), -1e30, f32)

    mxl_ref[...] = jnp.maximum(mxl_ref[...], jnp.max(el, axis=0, keepdims=True))
    mxr_ref[...] = jnp.maximum(mxr_ref[...], jnp.max(er, axis=0, keepdims=True))


def _dense1(xp, W, AL, AR):
    return pl.pallas_call(
        _dense1_body,
        grid=(NP // _BR,),
        in_specs=[pl.BlockSpec((_BR, HD), lambda i: (i, 0)),
                  pl.BlockSpec((HD, HD), lambda i: (0, 0)),
                  pl.BlockSpec((HD, 16), lambda i: (0, 0)),
                  pl.BlockSpec((HD, 16), lambda i: (0, 0))],
        out_specs=[pl.BlockSpec((_BR, HD), lambda i: (i, 0)),
                   pl.BlockSpec((_BR, 16), lambda i: (i, 0)),
                   pl.BlockSpec((_BR, 16), lambda i: (i, 0)),
                   pl.BlockSpec((1, 16), lambda i: (0, 0)),
                   pl.BlockSpec((1, 16), lambda i: (0, 0))],
        out_shape=[jax.ShapeDtypeStruct((NP, HD), f32),
                   jax.ShapeDtypeStruct((NP, 16), f32),
                   jax.ShapeDtypeStruct((NP, 16), f32),
                   jax.ShapeDtypeStruct((1, 16), f32),
                   jax.ShapeDtypeStruct((1, 16), f32)],
    )(xp, W, AL, AR)


def _dense2_body(p0_ref, p1_ref, d0_ref, d1_ref, rx_ref, b_ref, w_ref,
                 al_ref, ar_ref,
                 z_ref, el_ref, er_ref, mxl_ref, mxr_ref):
    i = pl.program_id(0)
    rd = 1.0 / jnp.maximum(d0_ref[...] + d1_ref[...], 1e-9)
    rde = jnp.dot(rd, rx_ref[...], preferred_element_type=f32,
                  precision=_HIGH)
    h = jnp.maximum((p0_ref[...] + p1_ref[...]) * rde + b_ref[...], 0.0)
    z = jnp.dot(h, w_ref[...], preferred_element_type=f32, precision=_HIGH)
    z_ref[...] = z
    el = jnp.dot(z, al_ref[...], preferred_element_type=f32, precision=_HIGH)
    er = jnp.dot(z, ar_ref[...], preferred_element_type=f32, precision=_HIGH)
    el_ref[...] = el
    er_ref[...] = er

    @pl.when(i == 0)
    def _():
        mxl_ref[...] = jnp.full((1, 16), -1e30, f32)
        mxr_ref[...] = jnp.full((1, 16), -1e30, f32)

    mxl_ref[...] = jnp.maximum(mxl_ref[...], jnp.max(el, axis=0, keepdims=True))
    mxr_ref[...] = jnp.maximum(mxr_ref[...], jnp.max(er, axis=0, keepdims=True))


def _dense2(h0, h1p, d0, d1, rexp, brow, W, AL, AR):
    return pl.pallas_call(
        _dense2_body,
        grid=(NP // _BR,),
        in_specs=[pl.BlockSpec((_BR, HD), lambda i: (i, 0)),
                  pl.BlockSpec((_BR, HD), lambda i: (i, 0)),
                  pl.BlockSpec((_BR, 16), lambda i: (i, 0)),
                  pl.BlockSpec((_BR, 16), lambda i: (i, 0)),
                  pl.BlockSpec((16, HD), lambda i: (0, 0)),
                  pl.BlockSpec((1, HD), lambda i: (0, 0)),
                  pl.BlockSpec((HD, HD), lambda i: (0, 0)),
                  pl.BlockSpec((HD, 16), lambda i: (0, 0)),
                  pl.BlockSpec((HD, 16), lambda i: (0, 0))],
        out_specs=[pl.BlockSpec((_BR, HD), lambda i: (i, 0)),
                   pl.BlockSpec((_BR, 16), lambda i: (i, 0)),
                   pl.BlockSpec((_BR, 16), lambda i: (i, 0)),
                   pl.BlockSpec((1, 16), lambda i: (0, 0)),
                   pl.BlockSpec((1, 16), lambda i: (0, 0))],
        out_shape=[jax.ShapeDtypeStruct((NP, HD), f32),
                   jax.ShapeDtypeStruct((NP, 16), f32),
                   jax.ShapeDtypeStruct((NP, 16), f32),
                   jax.ShapeDtypeStruct((1, 16), f32),
                   jax.ShapeDtypeStruct((1, 16), f32)],
    )(h0, h1p, d0, d1, rexp, brow, W, AL, AR)


def _final_body(w0_ref, w1_ref, z2_ref, rexp_ref, b2_ref, wc_ref, bc_ref,
                o_ref):
    w = w0_ref[...] + w1_ref[...]
    rows = lax.broadcasted_iota(i32, (NP, 1), 0)
    w = jnp.where(rows < N, w, 0.0)
    we = jnp.dot(w, rexp_ref[...], preferred_element_type=f32,
                 precision=_HIGH)
    s = jnp.sum(we * z2_ref[...], axis=0, keepdims=True)
    hg = s * (1.0 / N) + b2_ref[...]
    o_ref[...] = jnp.dot(hg, wc_ref[...], preferred_element_type=f32,
                         precision=_HIGH) + bc_ref[...]


def _final(wp, z2, rexp, b2row, Wc, bcrow):
    return pl.pallas_call(
        _final_body,
        out_shape=jax.ShapeDtypeStruct((1, C), f32),
    )(wp[:NP], wp[NP:], z2, rexp, b2row, Wc, bcrow)


@functools.partial(
    pl.kernel,
    out_type=[jax.ShapeDtypeStruct((EP, 16), f32),
              jax.ShapeDtypeStruct((2 * NP, 16), f32)],
    mesh=_sc_mesh(),
    compiler_params=pltpu.CompilerParams(use_tc_tiling_on_sc=False),
    scratch_types=[
        pltpu.VMEM((K,), i32),
        pltpu.VMEM((K,), i32),
        pltpu.VMEM((K, 16), f32),
        pltpu.VMEM((K, 16), f32),
        pltpu.VMEM((K, 16), f32),
        pltpu.VMEM((16,), f32),
        pltpu.VMEM((RPT, 16), f32),
        pltpu.VMEM_SHARED((NP, 16), f32),
        pltpu.VMEM_SHARED((NP, 16), f32),
        pltpu.VMEM_SHARED((NP, 16), f32),
        pltpu.SemaphoreType.DMA,
        pltpu.SemaphoreType.DMA,
    ],
)
def _edge_stats(src_hbm, dst_hbm, el_hbm, er_hbm, b_hbm,
                ee_hbm, dp_hbm,
                srcv, dstv, elv, erv, eev, bv, zbuf, dacc, elsh, ersh,
                sem1, sem2):
    c = lax.axis_index("c")
    s = lax.axis_index("s")
    wid = s * NC + c
    base = wid * EW

    pltpu.sync_copy(b_hbm, bv)
    pltpu.sync_copy(el_hbm.at[pl.ds(s * RPT, RPT)],
                    elsh.at[pl.ds(s * RPT, RPT)])
    pltpu.sync_copy(er_hbm.at[pl.ds(s * RPT, RPT)],
                    ersh.at[pl.ds(s * RPT, RPT)])

    def _zrow(j, carry):
        zbuf[j, :] = jnp.zeros((16,), f32)
        return carry

    lax.fori_loop(0, RPT, _zrow, 0)
    pltpu.sync_copy(zbuf, dacc.at[pl.ds(s * RPT, RPT)])
    plsc.subcore_barrier()

    bvec = bv[...]

    def _chunk(t, carry):
        off = base + t * K
        i1 = pltpu.async_copy(src_hbm.at[pl.ds(off, K)], srcv, sem1)
        i2 = pltpu.async_copy(dst_hbm.at[pl.ds(off, K)], dstv, sem2)
        i1.wait()
        i2.wait()
        g1 = pltpu.async_copy(elsh.at[srcv], elv, sem1)
        g2 = pltpu.async_copy(ersh.at[dstv], erv, sem2)
        g1.wait()
        g2.wait()

        def _row(r, rc):
            x = elv[r, :] + erv[r, :]
            e = jnp.maximum(x, 0.2 * x)
            eev[r, :] = jnp.exp(e - bvec)
            return rc

        lax.fori_loop(0, K, _row, 0)
        w1 = pltpu.async_copy(eev, ee_hbm.at[pl.ds(off, K)], sem1)
        pltpu.sync_copy(eev, dacc.at[dstv], add=True)
        w1.wait()
        return carry

    lax.fori_loop(0, NCHUNK, _chunk, 0)
    plsc.subcore_barrier()
    pltpu.sync_copy(dacc.at[pl.ds(s * RPT, RPT)],
                    dp_hbm.at[pl.ds(c * NP + s * RPT, RPT)])


@functools.partial(
    pl.kernel,
    out_type=jax.ShapeDtypeStruct((2 * NP, HD), f32),
    mesh=_sc_mesh(),
    compiler_params=pltpu.CompilerParams(use_tc_tiling_on_sc=False),
    scratch_types=[
        pltpu.VMEM((K,), i32),
        pltpu.VMEM((K,), i32),
        pltpu.VMEM((K, 16), f32),
        pltpu.VMEM((K, HD), f32),
        pltpu.VMEM_SHARED((NP, HD), f32),
        pltpu.SemaphoreType.DMA,
        pltpu.SemaphoreType.DMA,
    ],
)
def _aggregate(src_hbm, dst_hbm, ee_hbm, z_hbm,
               hp_hbm,
               srcv, dstv, eev, zrows, hacc, sem1, sem2):
    c = lax.axis_index("c")
    s = lax.axis_index("s")
    wid = s * NC + c
    base = wid * EW

    def _zrow(j, carry):
        zrows[j // 8, pl.ds((j % 8) * D, D)] = jnp.zeros((16,), f32)
        return carry

    lax.fori_loop(0, K * 8, _zrow, 0)

    def _zcp(j, carry):
        pltpu.sync_copy(zrows, hacc.at[pl.ds(s * RPT + j * K, K)])
        return carry

    lax.fori_loop(0, RPT // K, _zcp, 0)
    plsc.subcore_barrier()

    def _chunk(t, carry):
        off = base + t * K
        i1 = pltpu.async_copy(src_hbm.at[pl.ds(off, K)], srcv, sem1)
        i2 = pltpu.async_copy(dst_hbm.at[pl.ds(off, K)], dstv, sem2)
        i1.wait()
        i2.wait()
        g1 = pltpu.async_copy(z_hbm.at[srcv], zrows, sem1)
        g2 = pltpu.async_copy(ee_hbm.at[pl.ds(off, K)], eev, sem2)
        g2.wait()
        g1.wait()

        def _row(r, rc):
            av = eev[r, :]
            for h in range(H):
                zrows[r, pl.ds(h * D, D)] = zrows[r, pl.ds(h * D, D)] * av[h]
            return rc

        lax.fori_loop(0, K, _row, 0)
        pltpu.sync_copy(zrows, hacc.at[dstv], add=True)
        return carry

    lax.fori_loop(0, NCHUNK, _chunk, 0)
    plsc.subcore_barrier()

    def _out(j, carry):
        r0 = s * RPT + j * K
        pltpu.sync_copy(hacc.at[pl.ds(r0, K)],
                        hp_hbm.at[pl.ds(c * NP + r0, K)])
        return carry

    lax.fori_loop(0, RPT // K, _out, 0)


@functools.partial(
    pl.kernel,
    out_type=jax.ShapeDtypeStruct((2 * NP, 16), f32),
    mesh=_sc_mesh(),
    compiler_params=pltpu.CompilerParams(use_tc_tiling_on_sc=False),
    scratch_types=[
        pltpu.VMEM((K,), i32),
        pltpu.VMEM((K,), i32),
        pltpu.VMEM((K, 16), f32),
        pltpu.VMEM((K, 16), f32),
        pltpu.VMEM((RPT, 16), f32),
        pltpu.VMEM_SHARED((NP, 16), f32),
        pltpu.VMEM_SHARED((NP, 16), f32),
        pltpu.SemaphoreType.DMA,
        pltpu.SemaphoreType.DMA,
    ],
)
def _edge_weights(src_hbm, dst_hbm, ee_hbm, dp_hbm,
                  wp_hbm,
                  srcv, dstv, eev, rdv, zbuf, wacc, rdsh, sem1, sem2):
    c = lax.axis_index("c")
    s = lax.axis_index("s")
    wid = s * NC + c
    base = wid * EW

    def _rc(j, carry):
        r0 = s * RPT + j * K
        pltpu.sync_copy(dp_hbm.at[pl.ds(r0, K)], eev)
        pltpu.sync_copy(dp_hbm.at[pl.ds(NP + r0, K)], rdv)

        def _row(r, rc2):
            eev[r, :] = 1.0 / jnp.maximum(eev[r, :] + rdv[r, :], 1e-9)
            return rc2

        lax.fori_loop(0, K, _row, 0)
        pltpu.sync_copy(eev, rdsh.at[pl.ds(r0, K)])
        return carry

    lax.fori_loop(0, RPT // K, _rc, 0)

    def _zrow(j, carry):
        zbuf[j, :] = jnp.zeros((16,), f32)
        return carry

    lax.fori_loop(0, RPT, _zrow, 0)
    pltpu.sync_copy(zbuf, wacc.at[pl.ds(s * RPT, RPT)])
    plsc.subcore_barrier()

    def _chunk(t, carry):
        off = base + t * K
        i1 = pltpu.async_copy(src_hbm.at[pl.ds(off, K)], srcv, sem1)
        i2 = pltpu.async_copy(dst_hbm.at[pl.ds(off, K)], dstv, sem2)
        i1.wait()
        i2.wait()
        g1 = pltpu.async_copy(rdsh.at[dstv], rdv, sem1)
        g2 = pltpu.async_copy(ee_hbm.at[pl.ds(off, K)], eev, sem2)
        g2.wait()
        g1.wait()

        def _row(r, rc):
            eev[r, :] = eev[r, :] * rdv[r, :]
            return rc

        lax.fori_loop(0, K, _row, 0)
        pltpu.sync_copy(eev, wacc.at[srcv], add=True)
        return carry

    lax.fori_loop(0, NCHUNK, _chunk, 0)
    plsc.subcore_barrier()
    pltpu.sync_copy(wacc.at[pl.ds(s * RPT, RPT)],
                    wp_hbm.at[pl.ds(c * NP + s * RPT, RPT)])


def _amat(a):
    af = a.reshape(HD)
    cols = jnp.arange(HD) // D
    m = jnp.zeros((HD, 16), f32)
    m = m.at[jnp.arange(HD), cols].set(af)
    m = m.at[jnp.arange(HD), cols + 8].set(af)
    return m


def _lrelu(x):
    return jnp.maximum(x, 0.2 * x)


def kernel(feat, edge_index, W1, al1, ar1, b1, W2, al2, ar2, b2, Wc, bc):
    xp = jnp.zeros((NP, DIN), f32).at[:N].set(feat)
    pad_ids = N + (jnp.arange(EP - E, dtype=i32) % 16)
    srcp = jnp.concatenate([edge_index[0], pad_ids])
    dstp = jnp.concatenate([edge_index[1], pad_ids])
    AL1, AR1 = _amat(al1), _amat(ar1)
    AL2, AR2 = _amat(al2), _amat(ar2)
    cc = jnp.arange(HD)
    rexp = ((cc[None, :] // D) == (jnp.arange(16)[:, None] % 8)).astype(f32) * 0.5

    z1, el1, er1, mxl1, mxr1 = _dense1(xp, W1, AL1, AR1)
    b16_1 = _lrelu(mxl1 + mxr1).reshape(16)
    ee1, dp1 = _edge_stats(srcp, dstp, el1, er1, b16_1)
    h1p = _aggregate(srcp, dstp, ee1, z1)

    z2, el2, er2, mxl2, mxr2 = _dense2(
        h1p[:NP], h1p[NP:], dp1[:NP], dp1[NP:], rexp,
        b1.reshape(1, HD), W2, AL2, AR2)
    b16_2 = _lrelu(mxl2 + mxr2).reshape(16)
    ee2, dp2 = _edge_stats(srcp, dstp, el2, er2, b16_2)
    wp = _edge_weights(srcp, dstp, ee2, dp2)

    return _final(wp, z2, rexp, b2.reshape(1, HD), Wc, bc.reshape(1, C))
